# baseline (speedup 1.0000x reference)
import functools

import jax
import jax.numpy as jnp
from jax.experimental import pallas as pl
from jax.experimental.pallas import tpu as pltpu

_NUM_EXPERTS = 8
_TOP_K = 2
_D_MODEL = 768
_D_FF = 3072
_T = 2048
_FT = 1536


def _gelu_exact(v):
    return 0.5 * v * (1.0 + jax.lax.erf(v * 0.7071067811865476)).astype(v.dtype)


def _moe_kernel(x_ref, wg_ref, w1_ref, b1_ref, w2_ref, b2_ref, out_ref,
                xb_ref, w0_ref, w1s_ref):
    e = pl.program_id(0)
    f = pl.program_id(1)

    @pl.when((e == 0) & (f == 0))
    def _init():
        xv = x_ref[...]
        xb_ref[...] = xv.astype(jnp.bfloat16)
        logits = jnp.dot(xv, wg_ref[...], preferred_element_type=jnp.float32)
        l1 = jnp.max(logits, axis=1, keepdims=True)
        idx = jnp.argmax(logits, axis=1)[:, None]
        lanes = jax.lax.broadcasted_iota(jnp.int32, logits.shape, 1)
        l2 = jnp.max(jnp.where(lanes == idx, -jnp.inf, logits), axis=1,
                     keepdims=True)
        r = jnp.exp(l2 - l1)
        w0 = 1.0 / (1.0 + r)
        w0_ref[...] = w0.astype(jnp.bfloat16)
        w1s_ref[...] = (1.0 - w0).astype(jnp.bfloat16)
        bias = _TOP_K * jnp.sum(b2_ref[...], axis=0, keepdims=True)
        out_ref[...] = jnp.broadcast_to(bias, out_ref.shape)

    w1b = w1_ref[0].astype(jnp.bfloat16)
    w2b = w2_ref[0].astype(jnp.bfloat16)
    b1v = b1_ref[0].astype(jnp.bfloat16)
    two_b1 = b1v + b1v
    c = jnp.bfloat16(0.7071067811865476)
    half = jnp.bfloat16(0.5)
    nc = 8
    tc = _T // nc

    def _dot1(k):
        rows = slice(k * tc, (k + 1) * tc)
        return jnp.dot(xb_ref[rows, :], w1b,
                       preferred_element_type=jnp.float32).astype(jnp.bfloat16)

    z = _dot1(0)
    for k in range(nc):
        rows = slice(k * tc, (k + 1) * tc)
        z_next = _dot1(k + 1) if k + 1 < nc else None
        s = z + two_b1
        a0 = w0_ref[rows, :] * z + b1v
        a1 = s - a0
        p = a0 * jax.lax.erf(c * a0) + a1 * jax.lax.erf(c * a1)
        h = half * (s + p)
        out_ref[rows, :] += jnp.dot(h, w2b,
                                    preferred_element_type=jnp.float32)
        z = z_next


@functools.partial(jax.jit, static_argnames=())
def kernel(x, Wg, W1, b1, W2, b2):
    b, t, d = x.shape
    x2 = x.reshape(t, d)
    b1r = b1.reshape(_NUM_EXPERTS, 1, _D_FF)
    nf = _D_FF // _FT
    out = pl.pallas_call(
        _moe_kernel,
        grid=(_NUM_EXPERTS, nf),
        in_specs=[
            pl.BlockSpec((t, d), lambda e, f: (0, 0)),
            pl.BlockSpec((d, _NUM_EXPERTS), lambda e, f: (0, 0)),
            pl.BlockSpec((1, d, _FT), lambda e, f: (e, 0, f)),
            pl.BlockSpec((1, 1, _FT), lambda e, f: (e, 0, f)),
            pl.BlockSpec((1, _FT, d), lambda e, f: (e, f, 0)),
            pl.BlockSpec((_NUM_EXPERTS, d), lambda e, f: (0, 0)),
        ],
        out_specs=pl.BlockSpec((t, d), lambda e, f: (0, 0)),
        out_shape=jax.ShapeDtypeStruct((t, d), jnp.float32),
        scratch_shapes=[
            pltpu.VMEM((t, d), jnp.bfloat16),
            pltpu.VMEM((t, 1), jnp.bfloat16),
            pltpu.VMEM((t, 1), jnp.bfloat16),
        ],
        compiler_params=pltpu.CompilerParams(
            dimension_semantics=("arbitrary", "arbitrary"),
        ),
    )(x2, Wg, W1, b1r, W2, b2)
    return out.reshape(b, t, d)


# speedup vs baseline: 1.0041x; 1.0041x over previous
import functools

import jax
import jax.numpy as jnp
from jax.experimental import pallas as pl
from jax.experimental.pallas import tpu as pltpu

_NUM_EXPERTS = 8
_TOP_K = 2
_D_MODEL = 768
_D_FF = 3072
_T = 2048
_FT = 1536


def _gelu_exact(v):
    return 0.5 * v * (1.0 + jax.lax.erf(v * 0.7071067811865476)).astype(v.dtype)


def _moe_kernel(x_ref, wg_ref, w1_ref, b1_ref, w2_ref, b2_ref, out_ref,
                xb_ref, w0_ref, w1s_ref):
    e = pl.program_id(0)
    f = pl.program_id(1)

    @pl.when((e == 0) & (f == 0))
    def _init():
        xv = x_ref[...]
        xb_ref[...] = xv.astype(jnp.bfloat16)
        logits = jnp.dot(xv, wg_ref[...], preferred_element_type=jnp.float32)
        l1 = jnp.max(logits, axis=1, keepdims=True)
        idx = jnp.argmax(logits, axis=1)[:, None]
        lanes = jax.lax.broadcasted_iota(jnp.int32, logits.shape, 1)
        l2 = jnp.max(jnp.where(lanes == idx, -jnp.inf, logits), axis=1,
                     keepdims=True)
        r = jnp.exp(l2 - l1)
        w0 = 1.0 / (1.0 + r)
        w0_ref[...] = w0.astype(jnp.bfloat16)
        w1s_ref[...] = (1.0 - w0).astype(jnp.bfloat16)
        bias = _TOP_K * jnp.sum(b2_ref[...], axis=0, keepdims=True)
        out_ref[...] = jnp.broadcast_to(bias, out_ref.shape)

    w1b = w1_ref[0].astype(jnp.bfloat16)
    w2b = w2_ref[0].astype(jnp.bfloat16)
    b1v = b1_ref[0].astype(jnp.bfloat16)
    two_b1 = b1v + b1v
    c = jnp.bfloat16(0.7071067811865476)
    half = jnp.bfloat16(0.5)
    nc = 2
    tc = _T // nc

    def _dot1(k):
        rows = slice(k * tc, (k + 1) * tc)
        return jnp.dot(xb_ref[rows, :], w1b,
                       preferred_element_type=jnp.float32).astype(jnp.bfloat16)

    z = _dot1(0)
    for k in range(nc):
        rows = slice(k * tc, (k + 1) * tc)
        z_next = _dot1(k + 1) if k + 1 < nc else None
        s = z + two_b1
        a0 = w0_ref[rows, :] * z + b1v
        a1 = s - a0
        p = a0 * jax.lax.erf(c * a0) + a1 * jax.lax.erf(c * a1)
        h = half * (s + p)
        out_ref[rows, :] += jnp.dot(h, w2b,
                                    preferred_element_type=jnp.float32)
        z = z_next


@functools.partial(jax.jit, static_argnames=())
def kernel(x, Wg, W1, b1, W2, b2):
    b, t, d = x.shape
    x2 = x.reshape(t, d)
    b1r = b1.reshape(_NUM_EXPERTS, 1, _D_FF)
    nf = _D_FF // _FT
    out = pl.pallas_call(
        _moe_kernel,
        grid=(_NUM_EXPERTS, nf),
        in_specs=[
            pl.BlockSpec((t, d), lambda e, f: (0, 0)),
            pl.BlockSpec((d, _NUM_EXPERTS), lambda e, f: (0, 0)),
            pl.BlockSpec((1, d, _FT), lambda e, f: (e, 0, f)),
            pl.BlockSpec((1, 1, _FT), lambda e, f: (e, 0, f)),
            pl.BlockSpec((1, _FT, d), lambda e, f: (e, f, 0)),
            pl.BlockSpec((_NUM_EXPERTS, d), lambda e, f: (0, 0)),
        ],
        out_specs=pl.BlockSpec((t, d), lambda e, f: (0, 0)),
        out_shape=jax.ShapeDtypeStruct((t, d), jnp.float32),
        scratch_shapes=[
            pltpu.VMEM((t, d), jnp.bfloat16),
            pltpu.VMEM((t, 1), jnp.bfloat16),
            pltpu.VMEM((t, 1), jnp.bfloat16),
        ],
        compiler_params=pltpu.CompilerParams(
            dimension_semantics=("arbitrary", "arbitrary"),
        ),
    )(x2, Wg, W1, b1r, W2, b2)
    return out.reshape(b, t, d)


# speedup vs baseline: 1.0266x; 1.0224x over previous
import functools

import jax
import jax.numpy as jnp
from jax.experimental import pallas as pl
from jax.experimental.pallas import tpu as pltpu

_NUM_EXPERTS = 8
_TOP_K = 2
_D_MODEL = 768
_D_FF = 3072
_T = 2048
_FT = 1536


def _gelu_exact(v):
    return 0.5 * v * (1.0 + jax.lax.erf(v * 0.7071067811865476)).astype(v.dtype)


def _moe_kernel(x_ref, wg_ref, w1_ref, b1_ref, w2_ref, b2_ref, out_ref,
                xb_ref, w0_ref, w1s_ref):
    e = pl.program_id(0)
    f = pl.program_id(1)

    @pl.when((e == 0) & (f == 0))
    def _init():
        xv = x_ref[...]
        xb_ref[...] = xv.astype(jnp.bfloat16)
        logits = jnp.dot(xv, wg_ref[...], preferred_element_type=jnp.float32)
        l1 = jnp.max(logits, axis=1, keepdims=True)
        idx = jnp.argmax(logits, axis=1)[:, None]
        lanes = jax.lax.broadcasted_iota(jnp.int32, logits.shape, 1)
        l2 = jnp.max(jnp.where(lanes == idx, -jnp.inf, logits), axis=1,
                     keepdims=True)
        r = jnp.exp(l2 - l1)
        w0 = 1.0 / (1.0 + r)
        w0_ref[...] = w0.astype(jnp.bfloat16)
        w1s_ref[...] = (1.0 - w0).astype(jnp.bfloat16)
        bias = _TOP_K * jnp.sum(b2_ref[...], axis=0, keepdims=True)
        out_ref[...] = jnp.broadcast_to(bias, out_ref.shape)

    w1b = w1_ref[0].astype(jnp.bfloat16)
    w2b = w2_ref[0].astype(jnp.bfloat16)
    b1v = b1_ref[0].astype(jnp.bfloat16)
    two_b1 = b1v + b1v
    c = jnp.bfloat16(0.7071067811865476)
    half = jnp.bfloat16(0.5)
    nc = 4
    tc = _T // nc

    def _dot1(k):
        rows = slice(k * tc, (k + 1) * tc)
        return jnp.dot(xb_ref[rows, :], w1b,
                       preferred_element_type=jnp.float32).astype(jnp.bfloat16)

    z = _dot1(0)
    for k in range(nc):
        rows = slice(k * tc, (k + 1) * tc)
        z_next = _dot1(k + 1) if k + 1 < nc else None
        s = z + two_b1
        a0 = w0_ref[rows, :] * z + b1v
        a1 = s - a0
        p = a0 * jax.lax.erf(c * a0) + a1 * jax.lax.erf(c * a1)
        h = half * (s + p)
        out_ref[rows, :] += jnp.dot(h, w2b,
                                    preferred_element_type=jnp.float32)
        z = z_next


@functools.partial(jax.jit, static_argnames=())
def kernel(x, Wg, W1, b1, W2, b2):
    b, t, d = x.shape
    x2 = x.reshape(t, d)
    b1r = b1.reshape(_NUM_EXPERTS, 1, _D_FF)
    nf = _D_FF // _FT
    out = pl.pallas_call(
        _moe_kernel,
        grid=(_NUM_EXPERTS, nf),
        in_specs=[
            pl.BlockSpec((t, d), lambda e, f: (0, 0)),
            pl.BlockSpec((d, _NUM_EXPERTS), lambda e, f: (0, 0)),
            pl.BlockSpec((1, d, _FT), lambda e, f: (e, 0, f)),
            pl.BlockSpec((1, 1, _FT), lambda e, f: (e, 0, f)),
            pl.BlockSpec((1, _FT, d), lambda e, f: (e, f, 0)),
            pl.BlockSpec((_NUM_EXPERTS, d), lambda e, f: (0, 0)),
        ],
        out_specs=pl.BlockSpec((t, d), lambda e, f: (0, 0)),
        out_shape=jax.ShapeDtypeStruct((t, d), jnp.float32),
        scratch_shapes=[
            pltpu.VMEM((t, d), jnp.bfloat16),
            pltpu.VMEM((t, 1), jnp.bfloat16),
            pltpu.VMEM((t, 1), jnp.bfloat16),
        ],
        compiler_params=pltpu.CompilerParams(
            dimension_semantics=("arbitrary", "arbitrary"),
        ),
    )(x2, Wg, W1, b1r, W2, b2)
    return out.reshape(b, t, d)


# speedup vs baseline: 1.0336x; 1.0068x over previous
import functools

import jax
import jax.numpy as jnp
from jax.experimental import pallas as pl
from jax.experimental.pallas import tpu as pltpu

_NUM_EXPERTS = 8
_TOP_K = 2
_D_MODEL = 768
_D_FF = 3072
_T = 2048
_FT = 1536


def _moe_kernel(x_ref, wg_ref, w1_ref, b1_ref, w2_ref, b2_ref, out_ref,
                xb_ref, w0_ref):
    e = pl.program_id(0)
    f = pl.program_id(1)

    @pl.when((e == 0) & (f == 0))
    def _init():
        xv = x_ref[...]
        xb_ref[...] = xv.astype(jnp.bfloat16)
        logits = jnp.dot(xv, wg_ref[...], preferred_element_type=jnp.float32)
        l1 = jnp.max(logits, axis=1, keepdims=True)
        idx = jnp.argmax(logits, axis=1)[:, None]
        lanes = jax.lax.broadcasted_iota(jnp.int32, logits.shape, 1)
        l2 = jnp.max(jnp.where(lanes == idx, -jnp.inf, logits), axis=1,
                     keepdims=True)
        r = jnp.exp(l2 - l1)
        w0 = 1.0 / (1.0 + r)
        w0_ref[...] = w0.astype(jnp.bfloat16)
        bias = _TOP_K * jnp.sum(b2_ref[...], axis=0, keepdims=True)
        out_ref[...] = jnp.broadcast_to(bias, out_ref.shape)

    w1b = w1_ref[0].astype(jnp.bfloat16)
    w2b = w2_ref[0].astype(jnp.bfloat16)
    b1v = b1_ref[0].astype(jnp.bfloat16)
    two_b1 = b1v + b1v
    c = jnp.bfloat16(0.7071067811865476)
    half = jnp.bfloat16(0.5)
    nc = 4
    tc = _T // nc

    def _dot1(k):
        rows = slice(k * tc, (k + 1) * tc)
        return jnp.dot(xb_ref[rows, :], w1b,
                       preferred_element_type=jnp.float32).astype(jnp.bfloat16)

    z = _dot1(0)
    for k in range(nc):
        rows = slice(k * tc, (k + 1) * tc)
        z_next = _dot1(k + 1) if k + 1 < nc else None
        s = z + two_b1
        a0 = w0_ref[rows, :] * z + b1v
        a1 = s - a0
        p = a0 * jax.lax.erf(c * a0) + a1 * jax.lax.erf(c * a1)
        h = half * (s + p)
        out_ref[rows, :] += jnp.dot(h, w2b,
                                    preferred_element_type=jnp.float32)
        z = z_next


@functools.partial(jax.jit, static_argnames=())
def kernel(x, Wg, W1, b1, W2, b2):
    b, t, d = x.shape
    x2 = x.reshape(t, d)
    b1r = b1.reshape(_NUM_EXPERTS, 1, _D_FF)
    nf = _D_FF // _FT
    out = pl.pallas_call(
        _moe_kernel,
        grid=(_NUM_EXPERTS, nf),
        in_specs=[
            pl.BlockSpec((t, d), lambda e, f: (0, 0)),
            pl.BlockSpec((d, _NUM_EXPERTS), lambda e, f: (0, 0)),
            pl.BlockSpec((1, d, _FT), lambda e, f: (e, 0, f)),
            pl.BlockSpec((1, 1, _FT), lambda e, f: (e, 0, f)),
            pl.BlockSpec((1, _FT, d), lambda e, f: (e, f, 0)),
            pl.BlockSpec((_NUM_EXPERTS, d), lambda e, f: (0, 0)),
        ],
        out_specs=pl.BlockSpec((t, d), lambda e, f: (0, 0)),
        out_shape=jax.ShapeDtypeStruct((t, d), jnp.float32),
        scratch_shapes=[
            pltpu.VMEM((t, d), jnp.bfloat16),
            pltpu.VMEM((t, 1), jnp.bfloat16),
        ],
        compiler_params=pltpu.CompilerParams(
            dimension_semantics=("arbitrary", "arbitrary"),
        ),
    )(x2, Wg, W1, b1r, W2, b2)
    return out.reshape(b, t, d)
